# baseline (speedup 1.0000x reference)
import functools

import jax
import jax.numpy as jnp
from jax import lax
from jax.experimental import pallas as pl
from jax.experimental.pallas import tpu as pltpu
from jax.experimental.pallas import tpu_sc as plsc

T = 2048
D = 1024
H = 16
KV = 8
HD = 64
E = 8
K = 2
F = 512
THETA = 1000000.0
EPS = 1e-6
BT = 256
BM = 256
NBUF = T * K + E * BM
NB = NBUF // BM
DW = D // 2


def _rmsnorm(x, w):
    var = jnp.mean(x * x, axis=-1, keepdims=True)
    return x * jax.lax.rsqrt(var + EPS) * w


def _rope(x, cos, sin, nh):
    parts = []
    for h in range(nh):
        x1 = x[:, h * HD : h * HD + HD // 2]
        x2 = x[:, h * HD + HD // 2 : (h + 1) * HD]
        parts.append(x2)
        parts.append(x1)
    xs = jnp.concatenate(parts, axis=1)
    cosf = jnp.concatenate([cos] * nh, axis=1)
    sinf = jnp.concatenate([sin] * nh, axis=1)
    return x * cosf + xs * sinf


def _qkv_body(hid_ref, ln1_ref, cos_ref, sin_ref, wq_ref, wk_ref, wv_ref,
              bq_ref, bk_ref, bv_ref, qo_ref, ko_ref, vo_ref):
    x = hid_ref[...]
    xn = _rmsnorm(x, ln1_ref[...]).astype(jnp.bfloat16)
    cos = cos_ref[...]
    sin = sin_ref[...]
    q = jnp.dot(xn, wq_ref[...], preferred_element_type=jnp.float32) + bq_ref[...]
    k = jnp.dot(xn, wk_ref[...], preferred_element_type=jnp.float32) + bk_ref[...]
    v = jnp.dot(xn, wv_ref[...], preferred_element_type=jnp.float32) + bv_ref[...]
    qo_ref[...] = _rope(q, cos, sin, H).astype(jnp.bfloat16)
    ko_ref[...] = _rope(k, cos, sin, KV).astype(jnp.bfloat16)
    vo_ref[...] = v.astype(jnp.bfloat16)


def _attn_body(q_ref, k_ref, v_ref, o_ref, *, kext, qbase):
    qi = pl.program_id(0)
    rows = (qbase + qi) * BT + jax.lax.broadcasted_iota(jnp.int32, (BT, kext), 0)
    cols = jax.lax.broadcasted_iota(jnp.int32, (BT, kext), 1)
    causal = rows >= cols
    qb = q_ref[...]
    kb = k_ref[...]
    vb = v_ref[...]
    outs = []
    for h in range(H):
        qh = qb[:, h * HD : (h + 1) * HD]
        kvh = h // (H // KV)
        kh = kb[:, kvh * HD : (kvh + 1) * HD]
        vh = vb[:, kvh * HD : (kvh + 1) * HD]
        s = jax.lax.dot_general(qh, kh, (((1,), (1,)), ((), ())),
                                preferred_element_type=jnp.float32)
        s = s * (HD ** -0.5)
        s = jnp.where(causal, s, -1e30)
        m = jnp.max(s, axis=-1, keepdims=True)
        p = jnp.exp(s - m)
        denom = jnp.sum(p, axis=-1, keepdims=True)
        o = jnp.dot(p.astype(jnp.bfloat16), vh,
                    preferred_element_type=jnp.float32)
        outs.append((o / denom).astype(jnp.bfloat16))
    o_ref[...] = jnp.concatenate(outs, axis=1)


def _post_body(attn_ref, wo_ref, hid_ref, ln2_ref, wg_ref,
               h_ref, x2_ref, er_ref, rk_ref, wts_ref, cnt_ref, cnt_s):
    i = pl.program_id(0)
    a = attn_ref[...]
    ho = jnp.dot(a, wo_ref[...], preferred_element_type=jnp.float32)
    h = hid_ref[...] + ho
    h_ref[...] = h
    x2 = _rmsnorm(h, ln2_ref[...])
    x2_ref[...] = x2.astype(jnp.bfloat16)
    logits = jnp.dot(x2, wg_ref[...], preferred_element_type=jnp.float32)
    mx = jnp.max(logits, axis=-1, keepdims=True)
    pr = jnp.exp(logits - mx)
    pr = pr / jnp.sum(pr, axis=-1, keepdims=True)
    lanes = jax.lax.broadcasted_iota(jnp.int32, (BT, E), 1)
    m1 = jnp.max(pr, axis=-1, keepdims=True)
    idx1 = jnp.min(jnp.where(pr == m1, lanes, 127), axis=-1, keepdims=True)
    sel1 = lanes == idx1
    pr2 = jnp.where(sel1, -1.0, pr)
    m2 = jnp.max(pr2, axis=-1, keepdims=True)
    idx2 = jnp.min(jnp.where(pr2 == m2, lanes, 127), axis=-1, keepdims=True)
    sel2 = lanes == idx2
    msk = (sel1 | sel2).astype(jnp.bfloat16)
    rio = jax.lax.broadcasted_iota(jnp.int32, (BT, BT), 0)
    cio = jax.lax.broadcasted_iota(jnp.int32, (BT, BT), 1)
    ltri = (rio > cio).astype(jnp.bfloat16)
    rank_local = jnp.dot(ltri, msk, preferred_element_type=jnp.float32)

    @pl.when(i == 0)
    def _():
        cnt_s[...] = jnp.zeros((8, 128), jnp.float32)

    cnt_prev = cnt_s[0:1, 0:E]
    rank_g = rank_local + cnt_prev
    new_cnt = cnt_prev + jnp.sum(msk.astype(jnp.float32), axis=0, keepdims=True)
    cnt_s[0:1, 0:E] = new_cnt
    cnt_ref[...] = new_cnt
    sel1f = sel1.astype(jnp.float32)
    sel2f = sel2.astype(jnp.float32)
    rk1 = jnp.sum(rank_g * sel1f, axis=-1, keepdims=True)
    rk2 = jnp.sum(rank_g * sel2f, axis=-1, keepdims=True)
    rk_ref[...] = jnp.concatenate([rk1, rk2], axis=1)
    er_ref[...] = jnp.concatenate([idx1, idx2], axis=1)
    s12 = m1 + m2
    wts_ref[...] = jnp.concatenate([m1 / s12, m2 / s12], axis=1)


def _dest_body(cnt_ref, er_ref, rk_ref, dest_ref, bexp_ref):
    cnts = cnt_ref[...]
    pcnt = jnp.floor((cnts + (BM - 1)) / BM) * BM
    rio = jax.lax.broadcasted_iota(jnp.int32, (E, E), 0)
    cio = jax.lax.broadcasted_iota(jnp.int32, (E, E), 1)
    tri = (rio < cio).astype(jnp.float32)
    starts = jnp.dot(pcnt, tri, preferred_element_type=jnp.float32)
    er = er_ref[...]
    rk = rk_ref[...]
    dest = rk
    bcols = jax.lax.broadcasted_iota(jnp.int32, (1, 32), 1)
    bexp = jnp.full((1, 32), E, jnp.int32)
    for e in range(E):
        se = starts[0:1, e : e + 1]
        dest = dest + jnp.where(er == e,
                                jax.lax.broadcast_in_dim(se, (T, K), (0, 1)),
                                0.0)
        sb = (se / BM).astype(jnp.int32)
        nb = (pcnt[0:1, e : e + 1] / BM).astype(jnp.int32)
        inblk = (bcols >= sb) & (bcols < sb + nb)
        bexp = jnp.where(inblk, e, bexp)
    dest_ref[...] = dest.astype(jnp.int32)
    bexp_ref[...] = bexp


def _gmm_body(bexp_ref, xs_ref, wg_ref, wu_ref, wd_ref, ys_ref):
    be = bexp_ref[pl.program_id(0)]

    @pl.when(be < E)
    def _():
        x = xs_ref[...]
        w1 = wg_ref[0].astype(jnp.bfloat16)
        w2 = wu_ref[0].astype(jnp.bfloat16)
        w3 = wd_ref[0].astype(jnp.bfloat16)
        g = jnp.dot(x, w1, preferred_element_type=jnp.float32)
        u = jnp.dot(x, w2, preferred_element_type=jnp.float32)
        inter = (g * jax.lax.logistic(g) * u).astype(jnp.bfloat16)
        ys_ref[...] = jnp.dot(inter, w3,
                              preferred_element_type=jnp.float32).astype(jnp.bfloat16)


def _combine_body(h_ref, gp_ref, wts_ref, o_ref):
    h = h_ref[...]
    gp = gp_ref[...]
    w = wts_ref[...]
    y0 = gp[:, :D].astype(jnp.float32)
    y1 = gp[:, D:].astype(jnp.float32)
    o_ref[...] = h + w[:, 0:1] * y0 + w[:, 1:2] * y1


def _sc_dispatch(x2i, dflat):
    info = plsc.get_sparse_core_info()
    nc, ns = info.num_cores, info.num_subcores
    nw = nc * ns
    spw = (T * K) // nw
    tpw = T // nw
    mesh = plsc.VectorSubcoreMesh(core_axis_name="c", subcore_axis_name="s")

    @functools.partial(
        pl.kernel, mesh=mesh,
        out_type=jax.ShapeDtypeStruct((NBUF, DW), jnp.int32),
        scratch_types=[
            pltpu.VMEM((spw,), jnp.int32),
            pltpu.VMEM((spw,), jnp.int32),
            pltpu.VMEM((spw, DW), jnp.int32),
            pltpu.SemaphoreType.DMA,
            pltpu.SemaphoreType.DMA,
        ],
    )
    def k(x2_hbm, df_hbm, xs_hbm, tokidx_v, didx_v, rows_v, sem1, sem2):
        wid = lax.axis_index("s") * nc + lax.axis_index("c")
        base = wid * spw
        tb = wid * tpw
        for c in range(spw // 16):
            v = tb + ((lax.iota(jnp.int32, 16) + c * 16) >> 1)
            tokidx_v[pl.ds(c * 16, 16)] = v
        pltpu.sync_copy(df_hbm.at[pl.ds(base, spw)], didx_v)
        pltpu.async_copy(x2_hbm.at[tokidx_v], rows_v, sem1).wait()
        pltpu.async_copy(rows_v, xs_hbm.at[didx_v], sem2).wait()

    return k(x2i, dflat)


def _sc_collect(ysi, dflat):
    info = plsc.get_sparse_core_info()
    nc, ns = info.num_cores, info.num_subcores
    nw = nc * ns
    spw = (T * K) // nw
    mesh = plsc.VectorSubcoreMesh(core_axis_name="c", subcore_axis_name="s")

    @functools.partial(
        pl.kernel, mesh=mesh,
        out_type=jax.ShapeDtypeStruct((T * K, DW), jnp.int32),
        scratch_types=[
            pltpu.VMEM((spw,), jnp.int32),
            pltpu.VMEM((spw, DW), jnp.int32),
            pltpu.SemaphoreType.DMA,
        ],
    )
    def k(ys_hbm, df_hbm, g_hbm, didx_v, rows_v, sem):
        wid = lax.axis_index("s") * nc + lax.axis_index("c")
        base = wid * spw
        pltpu.sync_copy(df_hbm.at[pl.ds(base, spw)], didx_v)
        pltpu.async_copy(ys_hbm.at[didx_v], rows_v, sem).wait()
        pltpu.sync_copy(rows_v, g_hbm.at[pl.ds(base, spw)])

    return k(ysi, dflat)


def kernel(positions, hidden_states, ln1_w, ln2_w, Wq, bq, Wk, bk, Wv, bv,
           Wo, Wg, w_gate, w_up, w_down):
    f32 = jnp.float32
    bf16 = jnp.bfloat16
    half = HD // 2
    inv_freq = 1.0 / (THETA ** (jnp.arange(0, half, dtype=f32) / half))
    freqs = positions.astype(f32)[:, None] * inv_freq[None, :]
    c = jnp.cos(freqs)
    s = jnp.sin(freqs)
    cosA = jnp.concatenate([c, c], axis=1)
    sinA = jnp.concatenate([-s, s], axis=1)

    ln1 = ln1_w.reshape(1, D)
    ln2 = ln2_w.reshape(1, D)
    bq2 = bq.reshape(1, H * HD)
    bk2 = bk.reshape(1, KV * HD)
    bv2 = bv.reshape(1, KV * HD)
    wq_b = Wq.astype(bf16)
    wk_b = Wk.astype(bf16)
    wv_b = Wv.astype(bf16)
    wo_b = Wo.astype(bf16)

    nt = T // BT
    q, k, v = pl.pallas_call(
        _qkv_body,
        grid=(nt,),
        in_specs=[
            pl.BlockSpec((BT, D), lambda i: (i, 0)),
            pl.BlockSpec((1, D), lambda i: (0, 0)),
            pl.BlockSpec((BT, HD), lambda i: (i, 0)),
            pl.BlockSpec((BT, HD), lambda i: (i, 0)),
            pl.BlockSpec((D, H * HD), lambda i: (0, 0)),
            pl.BlockSpec((D, KV * HD), lambda i: (0, 0)),
            pl.BlockSpec((D, KV * HD), lambda i: (0, 0)),
            pl.BlockSpec((1, H * HD), lambda i: (0, 0)),
            pl.BlockSpec((1, KV * HD), lambda i: (0, 0)),
            pl.BlockSpec((1, KV * HD), lambda i: (0, 0)),
        ],
        out_specs=[
            pl.BlockSpec((BT, H * HD), lambda i: (i, 0)),
            pl.BlockSpec((BT, KV * HD), lambda i: (i, 0)),
            pl.BlockSpec((BT, KV * HD), lambda i: (i, 0)),
        ],
        out_shape=[
            jax.ShapeDtypeStruct((T, H * HD), bf16),
            jax.ShapeDtypeStruct((T, KV * HD), bf16),
            jax.ShapeDtypeStruct((T, KV * HD), bf16),
        ],
    )(hidden_states, ln1, cosA, sinA, wq_b, wk_b, wv_b, bq2, bk2, bv2)

    attn_parts = []
    GRP = 2
    for g in range(nt // GRP):
        qbase = g * GRP
        kext = (qbase + GRP) * BT
        part = pl.pallas_call(
            functools.partial(_attn_body, kext=kext, qbase=qbase),
            grid=(GRP,),
            in_specs=[
                pl.BlockSpec((BT, H * HD), lambda i, qb=qbase: (qb + i, 0)),
                pl.BlockSpec((kext, KV * HD), lambda i: (0, 0)),
                pl.BlockSpec((kext, KV * HD), lambda i: (0, 0)),
            ],
            out_specs=pl.BlockSpec((BT, H * HD), lambda i: (i, 0)),
            out_shape=jax.ShapeDtypeStruct((GRP * BT, H * HD), bf16),
        )(q, k, v)
        attn_parts.append(part)
    attn = jnp.concatenate(attn_parts, axis=0)

    h, x2b, er, rk, wts, cnt = pl.pallas_call(
        _post_body,
        grid=(nt,),
        in_specs=[
            pl.BlockSpec((BT, H * HD), lambda i: (i, 0)),
            pl.BlockSpec((H * HD, D), lambda i: (0, 0)),
            pl.BlockSpec((BT, D), lambda i: (i, 0)),
            pl.BlockSpec((1, D), lambda i: (0, 0)),
            pl.BlockSpec((D, E), lambda i: (0, 0)),
        ],
        out_specs=[
            pl.BlockSpec((BT, D), lambda i: (i, 0)),
            pl.BlockSpec((BT, D), lambda i: (i, 0)),
            pl.BlockSpec((BT, K), lambda i: (i, 0)),
            pl.BlockSpec((BT, K), lambda i: (i, 0)),
            pl.BlockSpec((BT, K), lambda i: (i, 0)),
            pl.BlockSpec((1, E), lambda i: (0, 0)),
        ],
        out_shape=[
            jax.ShapeDtypeStruct((T, D), f32),
            jax.ShapeDtypeStruct((T, D), bf16),
            jax.ShapeDtypeStruct((T, K), jnp.int32),
            jax.ShapeDtypeStruct((T, K), f32),
            jax.ShapeDtypeStruct((T, K), f32),
            jax.ShapeDtypeStruct((1, E), f32),
        ],
        scratch_shapes=[pltpu.VMEM((8, 128), f32)],
    )(attn, wo_b, hidden_states, ln2, Wg)

    dest, bexp = pl.pallas_call(
        _dest_body,
        grid=(1,),
        in_specs=[
            pl.BlockSpec((1, E), lambda i: (0, 0)),
            pl.BlockSpec((T, K), lambda i: (0, 0)),
            pl.BlockSpec((T, K), lambda i: (0, 0)),
        ],
        out_specs=[
            pl.BlockSpec((T, K), lambda i: (0, 0)),
            pl.BlockSpec((1, 32), lambda i: (0, 0)),
        ],
        out_shape=[
            jax.ShapeDtypeStruct((T, K), jnp.int32),
            jax.ShapeDtypeStruct((1, 32), jnp.int32),
        ],
    )(cnt, er, rk)

    dflat = dest.reshape(T * K)
    bexp1 = bexp.reshape(32)

    x2i = jax.lax.bitcast_convert_type(x2b.reshape(T, DW, 2), jnp.int32)
    xs_i = _sc_dispatch(x2i, dflat)
    xs = jax.lax.bitcast_convert_type(xs_i, bf16).reshape(NBUF, D)

    ys = pl.pallas_call(
        _gmm_body,
        grid_spec=pltpu.PrefetchScalarGridSpec(
            num_scalar_prefetch=1,
            grid=(NB,),
            in_specs=[
                pl.BlockSpec((BM, D), lambda b, be: (b, 0)),
                pl.BlockSpec((1, D, F),
                             lambda b, be: (jnp.minimum(be[b], E - 1), 0, 0)),
                pl.BlockSpec((1, D, F),
                             lambda b, be: (jnp.minimum(be[b], E - 1), 0, 0)),
                pl.BlockSpec((1, F, D),
                             lambda b, be: (jnp.minimum(be[b], E - 1), 0, 0)),
            ],
            out_specs=pl.BlockSpec((BM, D), lambda b, be: (b, 0)),
        ),
        out_shape=jax.ShapeDtypeStruct((NBUF, D), bf16),
    )(bexp1, xs, w_gate, w_up, w_down)

    ysi = jax.lax.bitcast_convert_type(ys.reshape(NBUF, DW, 2), jnp.int32)
    g_i = _sc_collect(ysi, dflat)
    gpair = jax.lax.bitcast_convert_type(g_i, bf16).reshape(T, K * D)

    out = pl.pallas_call(
        _combine_body,
        grid=(nt,),
        in_specs=[
            pl.BlockSpec((BT, D), lambda i: (i, 0)),
            pl.BlockSpec((BT, K * D), lambda i: (i, 0)),
            pl.BlockSpec((BT, K), lambda i: (i, 0)),
        ],
        out_specs=pl.BlockSpec((BT, D), lambda i: (i, 0)),
        out_shape=jax.ShapeDtypeStruct((T, D), f32),
    )(h, gpair, wts)

    return out


# speedup vs baseline: 9.4860x; 9.4860x over previous
import functools

import jax
import jax.numpy as jnp
from jax import lax
from jax.experimental import pallas as pl
from jax.experimental.pallas import tpu as pltpu
from jax.experimental.pallas import tpu_sc as plsc

T = 2048
D = 1024
H = 16
KV = 8
HD = 64
E = 8
K = 2
F = 512
THETA = 1000000.0
EPS = 1e-6
BT = 256
BM = 256
NBUF = T * K + E * BM
NB = NBUF // BM
DW = D // 2


def _rmsnorm(x, w):
    var = jnp.mean(x * x, axis=-1, keepdims=True)
    return x * jax.lax.rsqrt(var + EPS) * w


def _rope(x, cos, sin, nh):
    parts = []
    for h in range(nh):
        x1 = x[:, h * HD : h * HD + HD // 2]
        x2 = x[:, h * HD + HD // 2 : (h + 1) * HD]
        parts.append(x2)
        parts.append(x1)
    xs = jnp.concatenate(parts, axis=1)
    cosf = jnp.concatenate([cos] * nh, axis=1)
    sinf = jnp.concatenate([sin] * nh, axis=1)
    return x * cosf + xs * sinf


def _qkv_body(hid_ref, ln1_ref, cos_ref, sin_ref, wq_ref, wk_ref, wv_ref,
              bq_ref, bk_ref, bv_ref, qo_ref, ko_ref, vo_ref):
    x = hid_ref[...]
    xn = _rmsnorm(x, ln1_ref[...]).astype(jnp.bfloat16)
    cos = cos_ref[...]
    sin = sin_ref[...]
    q = jnp.dot(xn, wq_ref[...], preferred_element_type=jnp.float32) + bq_ref[...]
    k = jnp.dot(xn, wk_ref[...], preferred_element_type=jnp.float32) + bk_ref[...]
    v = jnp.dot(xn, wv_ref[...], preferred_element_type=jnp.float32) + bv_ref[...]
    qo_ref[...] = _rope(q, cos, sin, H).astype(jnp.bfloat16)
    ko_ref[...] = _rope(k, cos, sin, KV).astype(jnp.bfloat16)
    vo_ref[...] = v.astype(jnp.bfloat16)


def _attn_body(q_ref, k_ref, v_ref, o_ref, *, kext, qbase):
    qi = pl.program_id(0)
    rows = (qbase + qi) * BT + jax.lax.broadcasted_iota(jnp.int32, (BT, kext), 0)
    cols = jax.lax.broadcasted_iota(jnp.int32, (BT, kext), 1)
    causal = rows >= cols
    qb = q_ref[...]
    kb = k_ref[...]
    vb = v_ref[...]
    outs = []
    for h in range(H):
        qh = qb[:, h * HD : (h + 1) * HD]
        kvh = h // (H // KV)
        kh = kb[:, kvh * HD : (kvh + 1) * HD]
        vh = vb[:, kvh * HD : (kvh + 1) * HD]
        s = jax.lax.dot_general(qh, kh, (((1,), (1,)), ((), ())),
                                preferred_element_type=jnp.float32)
        s = s * (HD ** -0.5)
        s = jnp.where(causal, s, -1e30)
        m = jnp.max(s, axis=-1, keepdims=True)
        p = jnp.exp(s - m)
        denom = jnp.sum(p, axis=-1, keepdims=True)
        o = jnp.dot(p.astype(jnp.bfloat16), vh,
                    preferred_element_type=jnp.float32)
        outs.append((o / denom).astype(jnp.bfloat16))
    o_ref[...] = jnp.concatenate(outs, axis=1)


def _post_body(attn_ref, wo_ref, hid_ref, ln2_ref, wg_ref,
               h_ref, x2_ref, er_ref, rk_ref, wts_ref, cnt_ref, cnt_s):
    i = pl.program_id(0)
    a = attn_ref[...]
    ho = jnp.dot(a, wo_ref[...], preferred_element_type=jnp.float32)
    h = hid_ref[...] + ho
    h_ref[...] = h
    x2 = _rmsnorm(h, ln2_ref[...])
    x2_ref[...] = x2
    logits = jnp.dot(x2, wg_ref[...], preferred_element_type=jnp.float32)
    mx = jnp.max(logits, axis=-1, keepdims=True)
    pr = jnp.exp(logits - mx)
    pr = pr / jnp.sum(pr, axis=-1, keepdims=True)
    lanes = jax.lax.broadcasted_iota(jnp.int32, (BT, E), 1)
    m1 = jnp.max(pr, axis=-1, keepdims=True)
    idx1 = jnp.min(jnp.where(pr == m1, lanes, 127), axis=-1, keepdims=True)
    sel1 = lanes == idx1
    pr2 = jnp.where(sel1, -1.0, pr)
    m2 = jnp.max(pr2, axis=-1, keepdims=True)
    idx2 = jnp.min(jnp.where(pr2 == m2, lanes, 127), axis=-1, keepdims=True)
    sel2 = lanes == idx2
    msk = (sel1 | sel2).astype(jnp.bfloat16)
    rio = jax.lax.broadcasted_iota(jnp.int32, (BT, BT), 0)
    cio = jax.lax.broadcasted_iota(jnp.int32, (BT, BT), 1)
    ltri = (rio > cio).astype(jnp.bfloat16)
    rank_local = jnp.dot(ltri, msk, preferred_element_type=jnp.float32)

    @pl.when(i == 0)
    def _():
        cnt_s[...] = jnp.zeros((8, 128), jnp.float32)

    cnt_prev = cnt_s[0:1, 0:E]
    rank_g = rank_local + cnt_prev
    new_cnt = cnt_prev + jnp.sum(msk.astype(jnp.float32), axis=0, keepdims=True)
    cnt_s[0:1, 0:E] = new_cnt
    cnt_ref[...] = new_cnt
    sel1f = sel1.astype(jnp.float32)
    sel2f = sel2.astype(jnp.float32)
    rk1 = jnp.sum(rank_g * sel1f, axis=-1, keepdims=True)
    rk2 = jnp.sum(rank_g * sel2f, axis=-1, keepdims=True)
    rk_ref[...] = jnp.concatenate([rk1, rk2], axis=1)
    er_ref[...] = jnp.concatenate([idx1, idx2], axis=1)
    s12 = m1 + m2
    wts_ref[...] = jnp.concatenate([m1 / s12, m2 / s12], axis=1)


def _dest_body(cnt_ref, er_ref, rk_ref, dest_ref, bexp_ref):
    cnts = cnt_ref[...]
    pcnt = jnp.floor((cnts + (BM - 1)) / BM) * BM
    rio = jax.lax.broadcasted_iota(jnp.int32, (E, E), 0)
    cio = jax.lax.broadcasted_iota(jnp.int32, (E, E), 1)
    tri = (rio < cio).astype(jnp.float32)
    starts = jnp.dot(pcnt, tri, preferred_element_type=jnp.float32)
    er = er_ref[...]
    rk = rk_ref[...]
    dest = rk
    bcols = jax.lax.broadcasted_iota(jnp.int32, (1, 32), 1)
    bexp = jnp.full((1, 32), E, jnp.int32)
    for e in range(E):
        se = starts[0:1, e : e + 1]
        dest = dest + jnp.where(er == e,
                                jax.lax.broadcast_in_dim(se, (T, K), (0, 1)),
                                0.0)
        sb = (se / BM).astype(jnp.int32)
        nb = (pcnt[0:1, e : e + 1] / BM).astype(jnp.int32)
        inblk = (bcols >= sb) & (bcols < sb + nb)
        bexp = jnp.where(inblk, e, bexp)
    dest_ref[...] = dest.astype(jnp.int32)
    bexp_ref[...] = bexp


def _gmm_body(bexp_ref, xs_ref, wg_ref, wu_ref, wd_ref, ys_ref):
    be = bexp_ref[pl.program_id(0)]

    @pl.when(be < E)
    def _():
        x = xs_ref[...].astype(jnp.bfloat16)
        w1 = wg_ref[0].astype(jnp.bfloat16)
        w2 = wu_ref[0].astype(jnp.bfloat16)
        w3 = wd_ref[0].astype(jnp.bfloat16)
        g = jnp.dot(x, w1, preferred_element_type=jnp.float32)
        u = jnp.dot(x, w2, preferred_element_type=jnp.float32)
        inter = (g * jax.lax.logistic(g) * u).astype(jnp.bfloat16)
        ys_ref[...] = jnp.dot(inter, w3, preferred_element_type=jnp.float32)


def _combine_body(h_ref, g0_ref, g1_ref, wts_ref, o_ref):
    h = h_ref[...]
    w = wts_ref[...]
    o_ref[...] = h + w[:, 0:1] * g0_ref[...] + w[:, 1:2] * g1_ref[...]


def _sc_dispatch(x2i, dflat):
    info = plsc.get_sparse_core_info()
    nc, ns = info.num_cores, info.num_subcores
    nw = nc * ns
    spw = (T * K) // nw
    mesh = plsc.VectorSubcoreMesh(core_axis_name="c", subcore_axis_name="s")

    hw = spw // 2

    @functools.partial(
        pl.kernel, mesh=mesh,
        out_type=jax.ShapeDtypeStruct((NBUF, D), jnp.float32),
        scratch_types=[
            pltpu.VMEM((spw,), jnp.int32),
            pltpu.VMEM((hw, D), jnp.float32),
            pltpu.SemaphoreType.DMA,
        ],
    )
    def k(x2_hbm, df_hbm, xs_hbm, didx_v, rows_v, sem):
        wid = lax.axis_index("s") * nc + lax.axis_index("c")
        base = wid * spw
        tok_base = (wid * spw) % T
        pltpu.sync_copy(df_hbm.at[pl.ds(base, spw)], didx_v)
        for cch in range(2):
            pltpu.sync_copy(x2_hbm.at[pl.ds(tok_base + cch * hw, hw)], rows_v)
            pltpu.async_copy(rows_v, xs_hbm.at[didx_v.at[pl.ds(cch * hw, hw)]],
                             sem).wait()

    return k(x2i, dflat)


def _sc_collect(ysi, dflat):
    info = plsc.get_sparse_core_info()
    nc, ns = info.num_cores, info.num_subcores
    nw = nc * ns
    spw = (T * K) // nw
    mesh = plsc.VectorSubcoreMesh(core_axis_name="c", subcore_axis_name="s")

    hw = spw // 2

    @functools.partial(
        pl.kernel, mesh=mesh,
        out_type=jax.ShapeDtypeStruct((T * K, D), jnp.float32),
        scratch_types=[
            pltpu.VMEM((spw,), jnp.int32),
            pltpu.VMEM((hw, D), jnp.float32),
            pltpu.SemaphoreType.DMA,
        ],
    )
    def k(ys_hbm, df_hbm, g_hbm, didx_v, rows_v, sem):
        wid = lax.axis_index("s") * nc + lax.axis_index("c")
        base = wid * spw
        pltpu.sync_copy(df_hbm.at[pl.ds(base, spw)], didx_v)
        for cch in range(2):
            pltpu.async_copy(ys_hbm.at[didx_v.at[pl.ds(cch * hw, hw)]], rows_v,
                             sem).wait()
            pltpu.sync_copy(rows_v, g_hbm.at[pl.ds(base + cch * hw, hw)])

    return k(ysi, dflat)


def kernel(positions, hidden_states, ln1_w, ln2_w, Wq, bq, Wk, bk, Wv, bv,
           Wo, Wg, w_gate, w_up, w_down):
    f32 = jnp.float32
    bf16 = jnp.bfloat16
    half = HD // 2
    inv_freq = 1.0 / (THETA ** (jnp.arange(0, half, dtype=f32) / half))
    freqs = positions.astype(f32)[:, None] * inv_freq[None, :]
    c = jnp.cos(freqs)
    s = jnp.sin(freqs)
    cosA = jnp.concatenate([c, c], axis=1)
    sinA = jnp.concatenate([-s, s], axis=1)

    ln1 = ln1_w.reshape(1, D)
    ln2 = ln2_w.reshape(1, D)
    bq2 = bq.reshape(1, H * HD)
    bk2 = bk.reshape(1, KV * HD)
    bv2 = bv.reshape(1, KV * HD)
    wq_b = Wq.astype(bf16)
    wk_b = Wk.astype(bf16)
    wv_b = Wv.astype(bf16)
    wo_b = Wo.astype(bf16)

    nt = T // BT
    q, k, v = pl.pallas_call(
        _qkv_body,
        grid=(nt,),
        in_specs=[
            pl.BlockSpec((BT, D), lambda i: (i, 0)),
            pl.BlockSpec((1, D), lambda i: (0, 0)),
            pl.BlockSpec((BT, HD), lambda i: (i, 0)),
            pl.BlockSpec((BT, HD), lambda i: (i, 0)),
            pl.BlockSpec((D, H * HD), lambda i: (0, 0)),
            pl.BlockSpec((D, KV * HD), lambda i: (0, 0)),
            pl.BlockSpec((D, KV * HD), lambda i: (0, 0)),
            pl.BlockSpec((1, H * HD), lambda i: (0, 0)),
            pl.BlockSpec((1, KV * HD), lambda i: (0, 0)),
            pl.BlockSpec((1, KV * HD), lambda i: (0, 0)),
        ],
        out_specs=[
            pl.BlockSpec((BT, H * HD), lambda i: (i, 0)),
            pl.BlockSpec((BT, KV * HD), lambda i: (i, 0)),
            pl.BlockSpec((BT, KV * HD), lambda i: (i, 0)),
        ],
        out_shape=[
            jax.ShapeDtypeStruct((T, H * HD), bf16),
            jax.ShapeDtypeStruct((T, KV * HD), bf16),
            jax.ShapeDtypeStruct((T, KV * HD), bf16),
        ],
    )(hidden_states, ln1, cosA, sinA, wq_b, wk_b, wv_b, bq2, bk2, bv2)

    attn_parts = []
    GRP = 2
    for g in range(nt // GRP):
        qbase = g * GRP
        kext = (qbase + GRP) * BT
        part = pl.pallas_call(
            functools.partial(_attn_body, kext=kext, qbase=qbase),
            grid=(GRP,),
            in_specs=[
                pl.BlockSpec((BT, H * HD), lambda i, qb=qbase: (qb + i, 0)),
                pl.BlockSpec((kext, KV * HD), lambda i: (0, 0)),
                pl.BlockSpec((kext, KV * HD), lambda i: (0, 0)),
            ],
            out_specs=pl.BlockSpec((BT, H * HD), lambda i: (i, 0)),
            out_shape=jax.ShapeDtypeStruct((GRP * BT, H * HD), bf16),
        )(q, k, v)
        attn_parts.append(part)
    attn = jnp.concatenate(attn_parts, axis=0)

    h, x2b, er, rk, wts, cnt = pl.pallas_call(
        _post_body,
        grid=(nt,),
        in_specs=[
            pl.BlockSpec((BT, H * HD), lambda i: (i, 0)),
            pl.BlockSpec((H * HD, D), lambda i: (0, 0)),
            pl.BlockSpec((BT, D), lambda i: (i, 0)),
            pl.BlockSpec((1, D), lambda i: (0, 0)),
            pl.BlockSpec((D, E), lambda i: (0, 0)),
        ],
        out_specs=[
            pl.BlockSpec((BT, D), lambda i: (i, 0)),
            pl.BlockSpec((BT, D), lambda i: (i, 0)),
            pl.BlockSpec((BT, K), lambda i: (i, 0)),
            pl.BlockSpec((BT, K), lambda i: (i, 0)),
            pl.BlockSpec((BT, K), lambda i: (i, 0)),
            pl.BlockSpec((1, E), lambda i: (0, 0)),
        ],
        out_shape=[
            jax.ShapeDtypeStruct((T, D), f32),
            jax.ShapeDtypeStruct((T, D), f32),
            jax.ShapeDtypeStruct((T, K), jnp.int32),
            jax.ShapeDtypeStruct((T, K), f32),
            jax.ShapeDtypeStruct((T, K), f32),
            jax.ShapeDtypeStruct((1, E), f32),
        ],
        scratch_shapes=[pltpu.VMEM((8, 128), f32)],
    )(attn, wo_b, hidden_states, ln2, Wg)

    dest, bexp = pl.pallas_call(
        _dest_body,
        grid=(1,),
        in_specs=[
            pl.BlockSpec((1, E), lambda i: (0, 0)),
            pl.BlockSpec((T, K), lambda i: (0, 0)),
            pl.BlockSpec((T, K), lambda i: (0, 0)),
        ],
        out_specs=[
            pl.BlockSpec((T, K), lambda i: (0, 0)),
            pl.BlockSpec((1, 32), lambda i: (0, 0)),
        ],
        out_shape=[
            jax.ShapeDtypeStruct((T, K), jnp.int32),
            jax.ShapeDtypeStruct((1, 32), jnp.int32),
        ],
    )(cnt, er, rk)

    dflat = jnp.concatenate([dest[:, 0], dest[:, 1]], axis=0)
    bexp1 = bexp.reshape(32)

    xs = _sc_dispatch(x2b, dflat)

    ys = pl.pallas_call(
        _gmm_body,
        grid_spec=pltpu.PrefetchScalarGridSpec(
            num_scalar_prefetch=1,
            grid=(NB,),
            in_specs=[
                pl.BlockSpec((BM, D), lambda b, be: (b, 0)),
                pl.BlockSpec((1, D, F),
                             lambda b, be: (jnp.minimum(be[b], E - 1), 0, 0)),
                pl.BlockSpec((1, D, F),
                             lambda b, be: (jnp.minimum(be[b], E - 1), 0, 0)),
                pl.BlockSpec((1, F, D),
                             lambda b, be: (jnp.minimum(be[b], E - 1), 0, 0)),
            ],
            out_specs=pl.BlockSpec((BM, D), lambda b, be: (b, 0)),
        ),
        out_shape=jax.ShapeDtypeStruct((NBUF, D), f32),
    )(bexp1, xs, w_gate, w_up, w_down)

    gg = _sc_collect(ys, dflat)

    out = pl.pallas_call(
        _combine_body,
        grid=(nt,),
        in_specs=[
            pl.BlockSpec((BT, D), lambda i: (i, 0)),
            pl.BlockSpec((BT, D), lambda i: (i, 0)),
            pl.BlockSpec((BT, D), lambda i: (i + T // BT, 0)),
            pl.BlockSpec((BT, K), lambda i: (i, 0)),
        ],
        out_specs=pl.BlockSpec((BT, D), lambda i: (i, 0)),
        out_shape=jax.ShapeDtypeStruct((T, D), f32),
    )(h, gg, gg, wts)

    return out


# speedup vs baseline: 9.6264x; 1.0148x over previous
import functools

import jax
import jax.numpy as jnp
from jax.experimental import pallas as pl
from jax.experimental.pallas import tpu as pltpu

T = 2048
D = 1024
H = 16
KV = 8
HD = 64
E = 8
K = 2
F = 512
THETA = 1000000.0
EPS = 1e-6
BT = 256


def _rmsnorm(x, w):
    var = jnp.mean(x * x, axis=-1, keepdims=True)
    return x * jax.lax.rsqrt(var + EPS) * w


def _rope(x, cos, sin, nh):
    parts = []
    for h in range(nh):
        x1 = x[:, h * HD : h * HD + HD // 2]
        x2 = x[:, h * HD + HD // 2 : (h + 1) * HD]
        parts.append(x2)
        parts.append(x1)
    xs = jnp.concatenate(parts, axis=1)
    cosf = jnp.concatenate([cos] * nh, axis=1)
    sinf = jnp.concatenate([sin] * nh, axis=1)
    return x * cosf + xs * sinf


def _qkv_body(hid_ref, ln1_ref, cos_ref, sin_ref, wq_ref, wk_ref, wv_ref,
              bq_ref, bk_ref, bv_ref, qo_ref, ko_ref, vo_ref):
    x = hid_ref[...]
    xn = _rmsnorm(x, ln1_ref[...]).astype(jnp.bfloat16)
    cos = cos_ref[...]
    sin = sin_ref[...]
    q = jnp.dot(xn, wq_ref[...], preferred_element_type=jnp.float32) + bq_ref[...]
    k = jnp.dot(xn, wk_ref[...], preferred_element_type=jnp.float32) + bk_ref[...]
    v = jnp.dot(xn, wv_ref[...], preferred_element_type=jnp.float32) + bv_ref[...]
    qo_ref[...] = _rope(q, cos, sin, H).astype(jnp.bfloat16)
    ko_ref[...] = _rope(k, cos, sin, KV).astype(jnp.bfloat16)
    vo_ref[...] = v.astype(jnp.bfloat16)


def _attn_body(q_ref, k_ref, v_ref, o_ref, *, kext, qbase):
    qi = pl.program_id(0)
    rows = (qbase + qi) * BT + jax.lax.broadcasted_iota(jnp.int32, (BT, kext), 0)
    cols = jax.lax.broadcasted_iota(jnp.int32, (BT, kext), 1)
    causal = rows >= cols
    qb = q_ref[...]
    kb = k_ref[...]
    vb = v_ref[...]
    outs = []
    for h in range(H):
        qh = qb[:, h * HD : (h + 1) * HD]
        kvh = h // (H // KV)
        kh = kb[:, kvh * HD : (kvh + 1) * HD]
        vh = vb[:, kvh * HD : (kvh + 1) * HD]
        s = jax.lax.dot_general(qh, kh, (((1,), (1,)), ((), ())),
                                preferred_element_type=jnp.float32)
        s = s * (HD ** -0.5)
        s = jnp.where(causal, s, -1e30)
        m = jnp.max(s, axis=-1, keepdims=True)
        p = jnp.exp(s - m)
        denom = jnp.sum(p, axis=-1, keepdims=True)
        o = jnp.dot(p.astype(jnp.bfloat16), vh,
                    preferred_element_type=jnp.float32)
        outs.append((o / denom).astype(jnp.bfloat16))
    o_ref[...] = jnp.concatenate(outs, axis=1)


def _post_body(attn_ref, wo_ref, hid_ref, ln2_ref, wg_ref,
               h_ref, x2_ref, comb_ref):
    a = attn_ref[...]
    ho = jnp.dot(a, wo_ref[...], preferred_element_type=jnp.float32)
    h = hid_ref[...] + ho
    h_ref[...] = h
    x2 = _rmsnorm(h, ln2_ref[...])
    x2_ref[...] = x2.astype(jnp.bfloat16)
    logits = jnp.dot(x2, wg_ref[...], preferred_element_type=jnp.float32)
    mx = jnp.max(logits, axis=-1, keepdims=True)
    pr = jnp.exp(logits - mx)
    pr = pr / jnp.sum(pr, axis=-1, keepdims=True)
    lanes = jax.lax.broadcasted_iota(jnp.int32, (BT, E), 1)
    m1 = jnp.max(pr, axis=-1, keepdims=True)
    idx1 = jnp.min(jnp.where(pr == m1, lanes, 127), axis=-1, keepdims=True)
    sel1 = lanes == idx1
    pr2 = jnp.where(sel1, -1.0, pr)
    m2 = jnp.max(pr2, axis=-1, keepdims=True)
    idx2 = jnp.min(jnp.where(pr2 == m2, lanes, 127), axis=-1, keepdims=True)
    sel2 = lanes == idx2
    comb = (jnp.where(sel1, m1, 0.0) + jnp.where(sel2, m2, 0.0)) / (m1 + m2)
    comb_ref[...] = comb


def _moe_body(x2_ref, h_ref, comb_ref, wg_ref, wu_ref, wd_ref, o_ref, acc_s):
    e = pl.program_id(0)
    i = pl.program_id(1)
    rows = pl.ds(i * BT, BT)
    xb = x2_ref[rows, :]
    w1 = wg_ref[0].astype(jnp.bfloat16)
    w2 = wu_ref[0].astype(jnp.bfloat16)
    w3 = wd_ref[0].astype(jnp.bfloat16)
    g = jnp.dot(xb, w1, preferred_element_type=jnp.float32)
    u = jnp.dot(xb, w2, preferred_element_type=jnp.float32)
    inter = (g * jax.lax.logistic(g) * u).astype(jnp.bfloat16)
    y = jnp.dot(inter, w3, preferred_element_type=jnp.float32)
    cb = comb_ref[rows, :]
    lanes = jax.lax.broadcasted_iota(jnp.int32, (BT, E), 1)
    wsel = jnp.sum(jnp.where(lanes == e, cb, 0.0), axis=-1, keepdims=True)
    contrib = y * wsel

    @pl.when(e == 0)
    def _():
        acc_s[rows, :] = h_ref[rows, :] + contrib

    @pl.when(e != 0)
    def _():
        acc_s[rows, :] = acc_s[rows, :] + contrib

    o_ref[...] = acc_s[rows, :]


def kernel(positions, hidden_states, ln1_w, ln2_w, Wq, bq, Wk, bk, Wv, bv,
           Wo, Wg, w_gate, w_up, w_down):
    f32 = jnp.float32
    bf16 = jnp.bfloat16
    half = HD // 2
    inv_freq = 1.0 / (THETA ** (jnp.arange(0, half, dtype=f32) / half))
    freqs = positions.astype(f32)[:, None] * inv_freq[None, :]
    c = jnp.cos(freqs)
    s = jnp.sin(freqs)
    cosA = jnp.concatenate([c, c], axis=1)
    sinA = jnp.concatenate([-s, s], axis=1)

    ln1 = ln1_w.reshape(1, D)
    ln2 = ln2_w.reshape(1, D)
    bq2 = bq.reshape(1, H * HD)
    bk2 = bk.reshape(1, KV * HD)
    bv2 = bv.reshape(1, KV * HD)
    wq_b = Wq.astype(bf16)
    wk_b = Wk.astype(bf16)
    wv_b = Wv.astype(bf16)
    wo_b = Wo.astype(bf16)

    nt = T // BT
    q, k, v = pl.pallas_call(
        _qkv_body,
        grid=(nt,),
        in_specs=[
            pl.BlockSpec((BT, D), lambda i: (i, 0)),
            pl.BlockSpec((1, D), lambda i: (0, 0)),
            pl.BlockSpec((BT, HD), lambda i: (i, 0)),
            pl.BlockSpec((BT, HD), lambda i: (i, 0)),
            pl.BlockSpec((D, H * HD), lambda i: (0, 0)),
            pl.BlockSpec((D, KV * HD), lambda i: (0, 0)),
            pl.BlockSpec((D, KV * HD), lambda i: (0, 0)),
            pl.BlockSpec((1, H * HD), lambda i: (0, 0)),
            pl.BlockSpec((1, KV * HD), lambda i: (0, 0)),
            pl.BlockSpec((1, KV * HD), lambda i: (0, 0)),
        ],
        out_specs=[
            pl.BlockSpec((BT, H * HD), lambda i: (i, 0)),
            pl.BlockSpec((BT, KV * HD), lambda i: (i, 0)),
            pl.BlockSpec((BT, KV * HD), lambda i: (i, 0)),
        ],
        out_shape=[
            jax.ShapeDtypeStruct((T, H * HD), bf16),
            jax.ShapeDtypeStruct((T, KV * HD), bf16),
            jax.ShapeDtypeStruct((T, KV * HD), bf16),
        ],
    )(hidden_states, ln1, cosA, sinA, wq_b, wk_b, wv_b, bq2, bk2, bv2)

    attn_parts = []
    GRP = 2
    for g in range(nt // GRP):
        qbase = g * GRP
        kext = (qbase + GRP) * BT
        part = pl.pallas_call(
            functools.partial(_attn_body, kext=kext, qbase=qbase),
            grid=(GRP,),
            in_specs=[
                pl.BlockSpec((BT, H * HD), lambda i, qb=qbase: (qb + i, 0)),
                pl.BlockSpec((kext, KV * HD), lambda i: (0, 0)),
                pl.BlockSpec((kext, KV * HD), lambda i: (0, 0)),
            ],
            out_specs=pl.BlockSpec((BT, H * HD), lambda i: (i, 0)),
            out_shape=jax.ShapeDtypeStruct((GRP * BT, H * HD), bf16),
        )(q, k, v)
        attn_parts.append(part)
    attn = jnp.concatenate(attn_parts, axis=0)

    h, x2b, comb = pl.pallas_call(
        _post_body,
        grid=(nt,),
        in_specs=[
            pl.BlockSpec((BT, H * HD), lambda i: (i, 0)),
            pl.BlockSpec((H * HD, D), lambda i: (0, 0)),
            pl.BlockSpec((BT, D), lambda i: (i, 0)),
            pl.BlockSpec((1, D), lambda i: (0, 0)),
            pl.BlockSpec((D, E), lambda i: (0, 0)),
        ],
        out_specs=[
            pl.BlockSpec((BT, D), lambda i: (i, 0)),
            pl.BlockSpec((BT, D), lambda i: (i, 0)),
            pl.BlockSpec((BT, E), lambda i: (i, 0)),
        ],
        out_shape=[
            jax.ShapeDtypeStruct((T, D), f32),
            jax.ShapeDtypeStruct((T, D), bf16),
            jax.ShapeDtypeStruct((T, E), f32),
        ],
    )(attn, wo_b, hidden_states, ln2, Wg)

    out = pl.pallas_call(
        _moe_body,
        grid=(E, nt),
        in_specs=[
            pl.BlockSpec((T, D), lambda e, i: (0, 0)),
            pl.BlockSpec((T, D), lambda e, i: (0, 0)),
            pl.BlockSpec((T, E), lambda e, i: (0, 0)),
            pl.BlockSpec((1, D, F), lambda e, i: (e, 0, 0)),
            pl.BlockSpec((1, D, F), lambda e, i: (e, 0, 0)),
            pl.BlockSpec((1, F, D), lambda e, i: (e, 0, 0)),
        ],
        out_specs=pl.BlockSpec((BT, D), lambda e, i: (i, 0)),
        out_shape=jax.ShapeDtypeStruct((T, D), f32),
        scratch_shapes=[pltpu.VMEM((T, D), f32)],
    )(x2b, h, comb, w_gate, w_up, w_down)

    return out


# speedup vs baseline: 10.3814x; 1.0784x over previous
import functools

import jax
import jax.numpy as jnp
from jax.experimental import pallas as pl
from jax.experimental.pallas import tpu as pltpu

T = 2048
D = 1024
H = 16
KV = 8
HD = 64
E = 8
K = 2
F = 512
THETA = 1000000.0
EPS = 1e-6
BT = 256


def _rmsnorm(x, w):
    var = jnp.mean(x * x, axis=-1, keepdims=True)
    return x * jax.lax.rsqrt(var + EPS) * w


def _rope(x, cos, sin, nh):
    parts = []
    for h in range(nh):
        x1 = x[:, h * HD : h * HD + HD // 2]
        x2 = x[:, h * HD + HD // 2 : (h + 1) * HD]
        parts.append(x2)
        parts.append(x1)
    xs = jnp.concatenate(parts, axis=1)
    cosf = jnp.concatenate([cos] * nh, axis=1)
    sinf = jnp.concatenate([sin] * nh, axis=1)
    return x * cosf + xs * sinf


def _qkv_body(hid_ref, ln1_ref, cos_ref, sin_ref, wq_ref, wk_ref, wv_ref,
              bq_ref, bk_ref, bv_ref, qo_ref, ko_ref, vo_ref):
    x = hid_ref[...]
    xn = _rmsnorm(x, ln1_ref[...]).astype(jnp.bfloat16)
    cos = cos_ref[...]
    sin = sin_ref[...]
    q = jnp.dot(xn, wq_ref[...], preferred_element_type=jnp.float32) + bq_ref[...]
    k = jnp.dot(xn, wk_ref[...], preferred_element_type=jnp.float32) + bk_ref[...]
    v = jnp.dot(xn, wv_ref[...], preferred_element_type=jnp.float32) + bv_ref[...]
    qo_ref[...] = _rope(q, cos, sin, H).astype(jnp.bfloat16)
    ko_ref[...] = _rope(k, cos, sin, KV).astype(jnp.bfloat16)
    vo_ref[...] = v.astype(jnp.bfloat16)


def _attn_body(q_ref, k_ref, v_ref, o_ref, *, kext, qbase):
    qi = pl.program_id(0)
    rows = (qbase + qi) * BT + jax.lax.broadcasted_iota(jnp.int32, (BT, kext), 0)
    cols = jax.lax.broadcasted_iota(jnp.int32, (BT, kext), 1)
    causal = rows >= cols
    qb = q_ref[...]
    kb = k_ref[...]
    vb = v_ref[...]
    outs = []
    for h in range(H):
        qh = qb[:, h * HD : (h + 1) * HD]
        kvh = h // (H // KV)
        kh = kb[:, kvh * HD : (kvh + 1) * HD]
        vh = vb[:, kvh * HD : (kvh + 1) * HD]
        s = jax.lax.dot_general(qh, kh, (((1,), (1,)), ((), ())),
                                preferred_element_type=jnp.float32)
        s = s * (HD ** -0.5)
        s = jnp.where(causal, s, -1e30)
        m = jnp.max(s, axis=-1, keepdims=True)
        p = jnp.exp(s - m)
        denom = jnp.sum(p, axis=-1, keepdims=True)
        o = jnp.dot(p.astype(jnp.bfloat16), vh,
                    preferred_element_type=jnp.float32)
        outs.append((o / denom).astype(jnp.bfloat16))
    o_ref[...] = jnp.concatenate(outs, axis=1)


def _post_body(attn_ref, wo_ref, hid_ref, ln2_ref, wg_ref,
               h_ref, x2_ref, comb_ref):
    a = attn_ref[...]
    ho = jnp.dot(a, wo_ref[...], preferred_element_type=jnp.float32)
    h = hid_ref[...] + ho
    h_ref[...] = h
    x2 = _rmsnorm(h, ln2_ref[...])
    x2_ref[...] = x2.astype(jnp.bfloat16)
    logits = jnp.dot(x2, wg_ref[...], preferred_element_type=jnp.float32)
    mx = jnp.max(logits, axis=-1, keepdims=True)
    pr = jnp.exp(logits - mx)
    pr = pr / jnp.sum(pr, axis=-1, keepdims=True)
    lanes = jax.lax.broadcasted_iota(jnp.int32, (BT, E), 1)
    m1 = jnp.max(pr, axis=-1, keepdims=True)
    idx1 = jnp.min(jnp.where(pr == m1, lanes, 127), axis=-1, keepdims=True)
    sel1 = lanes == idx1
    pr2 = jnp.where(sel1, -1.0, pr)
    m2 = jnp.max(pr2, axis=-1, keepdims=True)
    idx2 = jnp.min(jnp.where(pr2 == m2, lanes, 127), axis=-1, keepdims=True)
    sel2 = lanes == idx2
    comb = (jnp.where(sel1, m1, 0.0) + jnp.where(sel2, m2, 0.0)) / (m1 + m2)
    comb_ref[...] = comb


def _moe_body(x2_ref, h_ref, comb_ref, wg_ref, wu_ref, wd_ref, o_ref):
    xb = x2_ref[...]
    acc = h_ref[...]
    comb = comb_ref[...]
    for e in range(E):
        g = jnp.dot(xb, wg_ref[e], preferred_element_type=jnp.float32)
        u = jnp.dot(xb, wu_ref[e], preferred_element_type=jnp.float32)
        inter = (g * jax.lax.logistic(g) * u).astype(jnp.bfloat16)
        y = jnp.dot(inter, wd_ref[e], preferred_element_type=jnp.float32)
        acc = acc + y * comb[:, e : e + 1]
    o_ref[...] = acc


def kernel(positions, hidden_states, ln1_w, ln2_w, Wq, bq, Wk, bk, Wv, bv,
           Wo, Wg, w_gate, w_up, w_down):
    f32 = jnp.float32
    bf16 = jnp.bfloat16
    half = HD // 2
    inv_freq = 1.0 / (THETA ** (jnp.arange(0, half, dtype=f32) / half))
    freqs = positions.astype(f32)[:, None] * inv_freq[None, :]
    c = jnp.cos(freqs)
    s = jnp.sin(freqs)
    cosA = jnp.concatenate([c, c], axis=1)
    sinA = jnp.concatenate([-s, s], axis=1)

    ln1 = ln1_w.reshape(1, D)
    ln2 = ln2_w.reshape(1, D)
    bq2 = bq.reshape(1, H * HD)
    bk2 = bk.reshape(1, KV * HD)
    bv2 = bv.reshape(1, KV * HD)
    wq_b = Wq.astype(bf16)
    wk_b = Wk.astype(bf16)
    wv_b = Wv.astype(bf16)
    wo_b = Wo.astype(bf16)
    wgate_b = w_gate.astype(bf16)
    wup_b = w_up.astype(bf16)
    wdown_b = w_down.astype(bf16)

    nt = T // BT
    q, k, v = pl.pallas_call(
        _qkv_body,
        grid=(nt,),
        in_specs=[
            pl.BlockSpec((BT, D), lambda i: (i, 0)),
            pl.BlockSpec((1, D), lambda i: (0, 0)),
            pl.BlockSpec((BT, HD), lambda i: (i, 0)),
            pl.BlockSpec((BT, HD), lambda i: (i, 0)),
            pl.BlockSpec((D, H * HD), lambda i: (0, 0)),
            pl.BlockSpec((D, KV * HD), lambda i: (0, 0)),
            pl.BlockSpec((D, KV * HD), lambda i: (0, 0)),
            pl.BlockSpec((1, H * HD), lambda i: (0, 0)),
            pl.BlockSpec((1, KV * HD), lambda i: (0, 0)),
            pl.BlockSpec((1, KV * HD), lambda i: (0, 0)),
        ],
        out_specs=[
            pl.BlockSpec((BT, H * HD), lambda i: (i, 0)),
            pl.BlockSpec((BT, KV * HD), lambda i: (i, 0)),
            pl.BlockSpec((BT, KV * HD), lambda i: (i, 0)),
        ],
        out_shape=[
            jax.ShapeDtypeStruct((T, H * HD), bf16),
            jax.ShapeDtypeStruct((T, KV * HD), bf16),
            jax.ShapeDtypeStruct((T, KV * HD), bf16),
        ],
    )(hidden_states, ln1, cosA, sinA, wq_b, wk_b, wv_b, bq2, bk2, bv2)

    attn_parts = []
    GRP = 1
    for g in range(nt // GRP):
        qbase = g * GRP
        kext = (qbase + GRP) * BT
        part = pl.pallas_call(
            functools.partial(_attn_body, kext=kext, qbase=qbase),
            grid=(GRP,),
            in_specs=[
                pl.BlockSpec((BT, H * HD), lambda i, qb=qbase: (qb + i, 0)),
                pl.BlockSpec((kext, KV * HD), lambda i: (0, 0)),
                pl.BlockSpec((kext, KV * HD), lambda i: (0, 0)),
            ],
            out_specs=pl.BlockSpec((BT, H * HD), lambda i: (i, 0)),
            out_shape=jax.ShapeDtypeStruct((GRP * BT, H * HD), bf16),
        )(q, k, v)
        attn_parts.append(part)
    attn = jnp.concatenate(attn_parts, axis=0)

    h, x2b, comb = pl.pallas_call(
        _post_body,
        grid=(nt,),
        in_specs=[
            pl.BlockSpec((BT, H * HD), lambda i: (i, 0)),
            pl.BlockSpec((H * HD, D), lambda i: (0, 0)),
            pl.BlockSpec((BT, D), lambda i: (i, 0)),
            pl.BlockSpec((1, D), lambda i: (0, 0)),
            pl.BlockSpec((D, E), lambda i: (0, 0)),
        ],
        out_specs=[
            pl.BlockSpec((BT, D), lambda i: (i, 0)),
            pl.BlockSpec((BT, D), lambda i: (i, 0)),
            pl.BlockSpec((BT, E), lambda i: (i, 0)),
        ],
        out_shape=[
            jax.ShapeDtypeStruct((T, D), f32),
            jax.ShapeDtypeStruct((T, D), bf16),
            jax.ShapeDtypeStruct((T, E), f32),
        ],
    )(attn, wo_b, hidden_states, ln2, Wg)

    out = pl.pallas_call(
        _moe_body,
        grid=(nt,),
        in_specs=[
            pl.BlockSpec((BT, D), lambda i: (i, 0)),
            pl.BlockSpec((BT, D), lambda i: (i, 0)),
            pl.BlockSpec((BT, E), lambda i: (i, 0)),
            pl.BlockSpec((E, D, F), lambda i: (0, 0, 0)),
            pl.BlockSpec((E, D, F), lambda i: (0, 0, 0)),
            pl.BlockSpec((E, F, D), lambda i: (0, 0, 0)),
        ],
        out_specs=pl.BlockSpec((BT, D), lambda i: (i, 0)),
        out_shape=jax.ShapeDtypeStruct((T, D), f32),
    )(x2b, h, comb, wgate_b, wup_b, wdown_b)

    return out
